# baseline (speedup 1.0000x reference)
import functools

import jax
import jax.numpy as jnp
from jax import lax
from jax.experimental import pallas as pl
from jax.experimental.pallas import tpu as pltpu
from jax.experimental.pallas import tpu_sc as plsc

_NUM_CORES = 2
_NUM_SUBCORES = 16
_NW = _NUM_CORES * _NUM_SUBCORES
_CHUNK = 32
_NROWBUF = 3
_GA = 2


@functools.partial(jax.jit, static_argnames=("b", "l", "d"))
def _sc_embed(tok, table, pe, *, b, l, d):
    span = l // _NW
    chunks_per_b = span // _CHUNK
    n_chunks = b * chunks_per_b
    nvec = d // 16

    mesh = plsc.VectorSubcoreMesh(
        core_axis_name="c", subcore_axis_name="s",
        num_cores=_NUM_CORES, num_subcores=_NUM_SUBCORES,
    )

    @functools.partial(
        pl.kernel,
        mesh=mesh,
        out_type=jax.ShapeDtypeStruct((b, l, d), jnp.float32),
        scratch_types=[
            pltpu.VMEM((b, span), jnp.int32),
            pltpu.VMEM((span, d), jnp.float32),
            [pltpu.VMEM((_CHUNK, d), jnp.float32) for _ in range(_NROWBUF)],
            pltpu.SemaphoreType.DMA,
            [pltpu.SemaphoreType.DMA for _ in range(_NROWBUF)],
            [pltpu.SemaphoreType.DMA for _ in range(_NROWBUF)],
        ],
    )
    def k(tok_hbm, table_hbm, pe_hbm, out_hbm,
          idx_v, pe_v, rows, sem_p, sem_g, sem_s):
        wid = lax.axis_index("s") * _NUM_CORES + lax.axis_index("c")
        l0 = wid * span

        def gather(c):
            bi, off = c // chunks_per_b, (c % chunks_per_b) * _CHUNK
            return pltpu.async_copy(
                table_hbm.at[idx_v.at[bi, pl.ds(off, _CHUNK)]],
                rows[c % _NROWBUF], sem_g[c % _NROWBUF])

        pend_tok = [
            pltpu.async_copy(tok_hbm.at[bb, pl.ds(l0, span)],
                             idx_v.at[bb], sem_p)
            for bb in range(b)
        ]
        pend_pe = pltpu.async_copy(pe_hbm.at[0, pl.ds(l0, span), :],
                                   pe_v, sem_p)
        for t in pend_tok:
            t.wait()
        pend_g = {c: gather(c) for c in range(min(_GA, n_chunks))}
        pend_s = {}
        pend_pe.wait()

        for c in range(n_chunks):
            rb = c % _NROWBUF
            bi, off = c // chunks_per_b, (c % chunks_per_b) * _CHUNK
            if c + _GA < n_chunks:
                if c + _GA - _NROWBUF >= 0:
                    pend_s.pop(c + _GA - _NROWBUF).wait()
                pend_g[c + _GA] = gather(c + _GA)
            pend_g.pop(c).wait()

            def add_row(r, _):
                for j in range(nvec):
                    sl = pl.ds(j * 16, 16)
                    plsc.addupdate(rows[rb].at[r, sl], pe_v[off + r, sl])
                return _

            lax.fori_loop(0, _CHUNK, add_row, 0, unroll=False)
            pend_s[c] = pltpu.async_copy(
                rows[rb], out_hbm.at[bi, pl.ds(l0 + off, _CHUNK), :],
                sem_s[rb])
        for c in sorted(pend_s):
            pend_s[c].wait()

    return k(tok, table, pe)


def kernel(tokens, table, pe):
    b, l = tokens.shape
    d = table.shape[1]
    return _sc_embed(tokens, table, pe, b=b, l=l, d=d)


# speedup vs baseline: 1.0680x; 1.0680x over previous
import functools

import jax
import jax.numpy as jnp
from jax import lax
from jax.experimental import pallas as pl
from jax.experimental.pallas import tpu as pltpu
from jax.experimental.pallas import tpu_sc as plsc

_NUM_CORES = 2
_NUM_SUBCORES = 16
_NW = _NUM_CORES * _NUM_SUBCORES
_CHUNK = 32
_NROWBUF = 3
_NPEBUF = 2


@functools.partial(jax.jit, static_argnames=("b", "l", "d"))
def _sc_embed(tok, table, pe, *, b, l, d):
    n = b * l
    per_w = n // _NW
    n_chunks = per_w // _CHUNK
    nvec = d // 16

    mesh = plsc.VectorSubcoreMesh(
        core_axis_name="c", subcore_axis_name="s",
        num_cores=_NUM_CORES, num_subcores=_NUM_SUBCORES,
    )

    @functools.partial(
        pl.kernel,
        mesh=mesh,
        out_type=jax.ShapeDtypeStruct((b, l, d), jnp.float32),
        scratch_types=[
            pltpu.VMEM((per_w,), jnp.int32),
            [pltpu.VMEM((_CHUNK, d), jnp.float32) for _ in range(_NROWBUF)],
            [pltpu.VMEM((_CHUNK, d), jnp.float32) for _ in range(_NPEBUF)],
            [pltpu.SemaphoreType.DMA for _ in range(_NROWBUF)],
            [pltpu.SemaphoreType.DMA for _ in range(_NPEBUF)],
            [pltpu.SemaphoreType.DMA for _ in range(_NROWBUF)],
        ],
    )
    def k(tok_hbm, table_hbm, pe_hbm, out_hbm,
          idx_v, rows, peb, sem_g, sem_p, sem_s):
        wid = lax.axis_index("s") * _NUM_CORES + lax.axis_index("c")
        bi = wid * per_w // l
        l0 = lax.rem(wid * per_w, l)

        def gather(c):
            return pltpu.async_copy(
                table_hbm.at[idx_v.at[pl.ds(c * _CHUNK, _CHUNK)]],
                rows[c % _NROWBUF], sem_g[c % _NROWBUF])

        def pe_load(c):
            return pltpu.async_copy(
                pe_hbm.at[0, pl.ds(l0 + c * _CHUNK, _CHUNK), :],
                peb[c % _NPEBUF], sem_p[c % _NPEBUF])

        pltpu.sync_copy(tok_hbm.at[bi, pl.ds(l0, per_w)], idx_v)
        pend_g = {c: gather(c) for c in range(min(2, n_chunks))}
        pend_p = {c: pe_load(c) for c in range(min(2, n_chunks))}
        pend_s = {}

        for c in range(n_chunks):
            rb, pb = c % _NROWBUF, c % _NPEBUF
            if c + 2 < n_chunks:
                if c >= 1:
                    pend_s.pop(c - 1).wait()
                pend_g[c + 2] = gather(c + 2)
            pend_g.pop(c).wait()
            pend_p.pop(c).wait()

            @plsc.parallel_loop(0, _CHUNK, 1)
            def add_row(r):
                for j in range(nvec):
                    sl = pl.ds(j * 16, 16)
                    plsc.addupdate(rows[rb].at[r, sl], peb[pb][r, sl])
            if c + 2 < n_chunks:
                pend_p[c + 2] = pe_load(c + 2)
            pend_s[c] = pltpu.async_copy(
                rows[rb], out_hbm.at[bi, pl.ds(l0 + c * _CHUNK, _CHUNK), :],
                sem_s[rb])
        for c in sorted(pend_s):
            pend_s[c].wait()

    return k(tok, table, pe)


def kernel(tokens, table, pe):
    b, l = tokens.shape
    d = table.shape[1]
    return _sc_embed(tokens, table, pe, b=b, l=l, d=d)
